# Initial kernel scaffold; baseline (speedup 1.0000x reference)
import jax
import jax.numpy as jnp
from jax.experimental import pallas as pl


def kernel(x, buffer, i):
    raise NotImplementedError("write your pallas kernel here")


# speedup vs baseline: 3.8072x; 3.8072x over previous
import functools

import jax
import jax.numpy as jnp
from jax import lax
from jax.experimental import pallas as pl
from jax.experimental.pallas import tpu as pltpu
from jax.experimental.pallas import tpu_sc as plsc

N = 16384
H = 200
L = 16


@functools.lru_cache(maxsize=None)
def _build():
    info = plsc.get_sparse_core_info()
    NC, NS = info.num_cores, info.num_subcores
    NW = NC * NS
    R = N // NW
    C = 128
    NCHUNK = R // C
    G = C // L

    mesh = plsc.VectorSubcoreMesh(core_axis_name="c", subcore_axis_name="s")

    @functools.partial(
        pl.kernel,
        mesh=mesh,
        out_type=jax.ShapeDtypeStruct((N * H,), jnp.float32),
        compiler_params=pltpu.CompilerParams(
            use_tc_tiling_on_sc=False, needs_layout_passes=False
        ),
        scratch_types=[
            pltpu.VMEM((R,), jnp.int32),
            pltpu.VMEM((R,), jnp.float32),
            pltpu.VMEM((C * H,), jnp.float32),
            pltpu.VMEM((C * H,), jnp.float32),
        ],
    )
    def sc_kernel(x_hbm, buf_hbm, i_hbm, out_hbm, i_v, x_v, in_v, out_v):
        wid = lax.axis_index("s") * NC + lax.axis_index("c")
        base = wid * R
        pltpu.sync_copy(i_hbm.at[pl.ds(base, R)], i_v)
        pltpu.sync_copy(x_hbm.at[pl.ds(base, R)], x_v)
        lane = lax.iota(jnp.int32, L)
        rowbases = [((g * L + lane) * H).astype(jnp.int32) for g in range(G)]
        for c in range(NCHUNK):
            fbase = (base + c * C) * H
            pltpu.sync_copy(buf_hbm.at[pl.ds(fbase, C * H)], in_v)
            srcs = []
            dsts = []
            for g in range(G):
                off = c * C + g * L
                ivec = i_v[pl.ds(off, L)]
                xvec = x_v[pl.ds(off, L)]
                plsc.store_scatter(in_v, [rowbases[g] + ivec], xvec)
                srcs.append(rowbases[g] + ivec)
                dsts.append(rowbases[g])

            def body(k, carry):
                srcs, dsts = carry
                nsrcs, ndsts = [], []
                for g in range(G):
                    vals = plsc.load_gather(in_v, [srcs[g]])
                    plsc.store_scatter(out_v, [dsts[g]], vals)
                    s = srcs[g] - 1
                    nsrcs.append(jnp.where(s < rowbases[g], s + H, s))
                    ndsts.append(dsts[g] + 1)
                return tuple(nsrcs), tuple(ndsts)

            lax.fori_loop(0, H, body, (tuple(srcs), tuple(dsts)))
            pltpu.sync_copy(out_v, out_hbm.at[pl.ds(fbase, C * H)])

    return sc_kernel


def kernel(x, buffer, i):
    out = _build()(x.reshape(-1), buffer.reshape(-1), i.astype(jnp.int32))
    return out.reshape(N, H)


# speedup vs baseline: 3.8096x; 1.0006x over previous
import functools

import jax
import jax.numpy as jnp
from jax import lax
from jax.experimental import pallas as pl
from jax.experimental.pallas import tpu as pltpu
from jax.experimental.pallas import tpu_sc as plsc

N = 16384
H = 200
L = 16


@functools.lru_cache(maxsize=None)
def _build():
    info = plsc.get_sparse_core_info()
    NC, NS = info.num_cores, info.num_subcores
    NW = NC * NS
    R = N // NW
    C = 128
    NCHUNK = R // C
    G = C // L

    mesh = plsc.VectorSubcoreMesh(core_axis_name="c", subcore_axis_name="s")

    @functools.partial(
        pl.kernel,
        mesh=mesh,
        out_type=jax.ShapeDtypeStruct((N, H), jnp.float32),
        compiler_params=pltpu.CompilerParams(
            use_tc_tiling_on_sc=False, needs_layout_passes=False
        ),
        scratch_types=[
            pltpu.VMEM((R,), jnp.int32),
            pltpu.VMEM((R,), jnp.float32),
            pltpu.VMEM((C, H), jnp.float32),
            pltpu.VMEM((C, H), jnp.float32),
        ],
    )
    def sc_kernel(x_hbm, buf_hbm, i_hbm, out_hbm, i_v, x_v, in_v, out_v):
        wid = lax.axis_index("s") * NC + lax.axis_index("c")
        base = wid * R
        pltpu.sync_copy(i_hbm.at[pl.ds(base, R)], i_v)
        pltpu.sync_copy(x_hbm.at[pl.ds(base, R)], x_v)
        lane = lax.iota(jnp.int32, L)
        rows = [g * L + lane for g in range(G)]
        zero = jnp.zeros((L,), jnp.int32)
        for c in range(NCHUNK):
            rbase = base + c * C
            pltpu.sync_copy(buf_hbm.at[pl.ds(rbase, C)], in_v)
            srcs = []
            for g in range(G):
                off = c * C + g * L
                ivec = i_v[pl.ds(off, L)]
                xvec = x_v[pl.ds(off, L)]
                plsc.store_scatter(in_v, [rows[g], ivec], xvec)
                srcs.append(ivec)

            def body(k, carry):
                srcs, dst = carry
                nsrcs = []
                for g in range(G):
                    vals = plsc.load_gather(in_v, [rows[g], srcs[g]])
                    plsc.store_scatter(out_v, [rows[g], dst], vals)
                    s = srcs[g] - 1
                    nsrcs.append(jnp.where(s < 0, s + H, s))
                return tuple(nsrcs), dst + 1

            lax.fori_loop(0, H, body, (tuple(srcs), zero))
            pltpu.sync_copy(out_v, out_hbm.at[pl.ds(rbase, C)])

    return sc_kernel


def kernel(x, buffer, i):
    return _build()(x.reshape(-1), buffer, i.astype(jnp.int32))


# speedup vs baseline: 4.0943x; 1.0747x over previous
import functools

import jax
import jax.numpy as jnp
from jax import lax
from jax.experimental import pallas as pl
from jax.experimental.pallas import tpu as pltpu
from jax.experimental.pallas import tpu_sc as plsc

N = 16384
H = 200
L = 16


@functools.lru_cache(maxsize=None)
def _build():
    info = plsc.get_sparse_core_info()
    NC, NS = info.num_cores, info.num_subcores
    NW = NC * NS
    R = N // NW
    C = 128
    NCHUNK = R // C
    G = C // L

    mesh = plsc.VectorSubcoreMesh(core_axis_name="c", subcore_axis_name="s")

    @functools.partial(
        pl.kernel,
        mesh=mesh,
        out_type=jax.ShapeDtypeStruct((N, H), jnp.float32),
        compiler_params=pltpu.CompilerParams(needs_layout_passes=False),
        scratch_types=[
            pltpu.VMEM((R,), jnp.int32),
            pltpu.VMEM((R,), jnp.float32),
            pltpu.VMEM((C, H), jnp.float32),
            pltpu.VMEM((C, H), jnp.float32),
        ],
    )
    def sc_kernel(x_hbm, buf_hbm, i_hbm, out_hbm, i_v, x_v, in_v, out_v):
        wid = lax.axis_index("s") * NC + lax.axis_index("c")
        base = wid * R
        pltpu.sync_copy(i_hbm.at[pl.ds(base, R)], i_v)
        pltpu.sync_copy(x_hbm.at[pl.ds(base, R)], x_v)
        lane = lax.iota(jnp.int32, L)
        rows = [g * L + lane for g in range(G)]
        zero = jnp.zeros((L,), jnp.int32)
        for c in range(NCHUNK):
            rbase = base + c * C
            pltpu.sync_copy(buf_hbm.at[pl.ds(rbase, C)], in_v)
            srcs = []
            for g in range(G):
                off = c * C + g * L
                ivec = i_v[pl.ds(off, L)]
                xvec = x_v[pl.ds(off, L)]
                plsc.store_scatter(in_v, [rows[g], ivec], xvec)
                srcs.append(ivec)

            def body(k, carry):
                srcs, dst = carry
                nsrcs = []
                for g in range(G):
                    vals = plsc.load_gather(in_v, [rows[g], srcs[g]])
                    plsc.store_scatter(out_v, [rows[g], dst], vals)
                    s = srcs[g] - 1
                    nsrcs.append(jnp.where(s < 0, s + H, s))
                return tuple(nsrcs), dst + 1

            lax.fori_loop(0, H, body, (tuple(srcs), zero))
            pltpu.sync_copy(out_v, out_hbm.at[pl.ds(rbase, C)])

    return sc_kernel


def kernel(x, buffer, i):
    return _build()(x.reshape(-1), buffer, i.astype(jnp.int32))


# speedup vs baseline: 12.5871x; 3.0743x over previous
import functools

import jax
import jax.numpy as jnp
from jax import lax
from jax.experimental import pallas as pl
from jax.experimental.pallas import tpu as pltpu
from jax.experimental.pallas import tpu_sc as plsc

N = 16384
H = 200
L = 16
W = 128


@functools.lru_cache(maxsize=None)
def _build():
    info = plsc.get_sparse_core_info()
    NC, NS = info.num_cores, info.num_subcores
    NW = NC * NS
    R = N // NW
    TCOLS = R // W
    G = W // L

    mesh = plsc.VectorSubcoreMesh(core_axis_name="c", subcore_axis_name="s")

    @functools.partial(
        pl.kernel,
        mesh=mesh,
        out_type=jax.ShapeDtypeStruct((H, N), jnp.float32),
        compiler_params=pltpu.CompilerParams(needs_layout_passes=False),
        scratch_types=[
            pltpu.VMEM((R,), jnp.int32),
            pltpu.VMEM((R,), jnp.float32),
            pltpu.VMEM((H, W), jnp.float32),
            pltpu.VMEM((H, W), jnp.float32),
        ],
    )
    def sc_kernel(x_hbm, bufT_hbm, i_hbm, outT_hbm, i_v, x_v, in_v, out_v):
        wid = lax.axis_index("s") * NC + lax.axis_index("c")
        base = wid * R
        pltpu.sync_copy(i_hbm.at[pl.ds(base, R)], i_v)
        pltpu.sync_copy(x_hbm.at[pl.ds(base, R)], x_v)
        lane = lax.iota(jnp.int32, L)
        for t in range(TCOLS):
            n0 = base + t * W
            pltpu.sync_copy(bufT_hbm.at[:, pl.ds(n0, W)], in_v)
            nls = []
            hs = []
            for g in range(G):
                off = t * W + g * L
                ivec = i_v[pl.ds(off, L)]
                xvec = x_v[pl.ds(off, L)]
                nl = g * L + lane
                plsc.store_scatter(in_v, [ivec, nl], xvec)
                nls.append(nl)
                hs.append(ivec)

            def body(k2, carry):
                hs, kv = carry
                for _ in range(2):
                    vals = [
                        plsc.load_gather(in_v, [hs[g], nls[g]])
                        for g in range(G)
                    ]
                    for g in range(G):
                        plsc.store_scatter(out_v, [kv, nls[g]], vals[g])
                    nhs = []
                    for g in range(G):
                        h = hs[g] - 1
                        nhs.append(jnp.where(h < 0, h + H, h))
                    hs = tuple(nhs)
                    kv = kv + 1
                return hs, kv

            zero = jnp.zeros((L,), jnp.int32)
            lax.fori_loop(0, H // 2, body, (tuple(hs), zero))
            pltpu.sync_copy(out_v, outT_hbm.at[:, pl.ds(n0, W)])

    return sc_kernel


def kernel(x, buffer, i):
    out_t = _build()(x.reshape(-1), buffer.T, i.astype(jnp.int32))
    return out_t.T


# speedup vs baseline: 15.5275x; 1.2336x over previous
import functools

import jax
import jax.numpy as jnp
from jax import lax
from jax.experimental import pallas as pl
from jax.experimental.pallas import tpu as pltpu
from jax.experimental.pallas import tpu_sc as plsc

N = 16384
H = 200
L = 16
W = 128


@functools.lru_cache(maxsize=None)
def _build():
    info = plsc.get_sparse_core_info()
    NC, NS = info.num_cores, info.num_subcores
    NW = NC * NS
    R = N // NW
    TCOLS = R // W
    G = W // L

    mesh = plsc.VectorSubcoreMesh(core_axis_name="c", subcore_axis_name="s")

    @functools.partial(
        pl.kernel,
        mesh=mesh,
        out_type=jax.ShapeDtypeStruct((H, N), jnp.float32),
        compiler_params=pltpu.CompilerParams(needs_layout_passes=False),
        scratch_types=[
            pltpu.VMEM((R,), jnp.int32),
            pltpu.VMEM((R,), jnp.float32),
            pltpu.VMEM((H, W), jnp.float32),
            pltpu.VMEM((H, W), jnp.float32),
            pltpu.VMEM((H, W), jnp.float32),
            pltpu.VMEM((H, W), jnp.float32),
            pltpu.SemaphoreType.DMA,
            pltpu.SemaphoreType.DMA,
            pltpu.SemaphoreType.DMA,
            pltpu.SemaphoreType.DMA,
        ],
    )
    def sc_kernel(
        x_hbm, bufT_hbm, i_hbm, outT_hbm,
        i_v, x_v, in0, in1, out0, out1, is0, is1, os0, os1,
    ):
        wid = lax.axis_index("s") * NC + lax.axis_index("c")
        base = wid * R
        pltpu.sync_copy(i_hbm.at[pl.ds(base, R)], i_v)
        pltpu.sync_copy(x_hbm.at[pl.ds(base, R)], x_v)
        lane = lax.iota(jnp.int32, L)
        ins, outs = [in0, in1], [out0, out1]
        isems, osems = [is0, is1], [os0, os1]

        def copy_in(t, dst, sem):
            return pltpu.make_async_copy(
                bufT_hbm.at[:, pl.ds(base + t * W, W)], dst, sem
            )

        def copy_out(t, src, sem):
            return pltpu.make_async_copy(
                src, outT_hbm.at[:, pl.ds(base + t * W, W)], sem
            )

        def compute(t, in_ref, out_ref):
            nls = []
            hs = []
            for g in range(G):
                off = t * W + g * L
                ivec = i_v[pl.ds(off, L)]
                xvec = x_v[pl.ds(off, L)]
                nl = g * L + lane
                plsc.store_scatter(in_ref, [ivec, nl], xvec)
                nls.append(nl)
                hs.append(ivec)

            def body(k2, hs):
                for step in range(2):
                    k = k2 * 2 + step
                    vals = [
                        plsc.load_gather(in_ref, [hs[g], nls[g]])
                        for g in range(G)
                    ]
                    for g in range(G):
                        out_ref[k, pl.ds(g * L, L)] = vals[g]
                    nhs = []
                    for g in range(G):
                        h = hs[g] - 1
                        nhs.append(jnp.where(h < 0, h + H, h))
                    hs = tuple(nhs)
                return hs

            lax.fori_loop(0, H // 2, body, tuple(hs))

        copy_in(0, ins[0], isems[0]).start()
        for t in range(TCOLS):
            b = t % 2
            if t + 1 < TCOLS:
                copy_in(t + 1, ins[1 - b], isems[1 - b]).start()
            copy_in(t, ins[b], isems[b]).wait()
            if t >= 2:
                copy_out(t - 2, outs[b], osems[b]).wait()
            compute(t, ins[b], outs[b])
            copy_out(t, outs[b], osems[b]).start()
        copy_out(TCOLS - 2, outs[(TCOLS - 2) % 2], osems[(TCOLS - 2) % 2]).wait()
        copy_out(TCOLS - 1, outs[(TCOLS - 1) % 2], osems[(TCOLS - 1) % 2]).wait()

    return sc_kernel


def kernel(x, buffer, i):
    out_t = _build()(x.reshape(-1), buffer.T, i.astype(jnp.int32))
    return out_t.T


# speedup vs baseline: 15.6447x; 1.0075x over previous
import functools

import jax
import jax.numpy as jnp
from jax import lax
from jax.experimental import pallas as pl
from jax.experimental.pallas import tpu as pltpu
from jax.experimental.pallas import tpu_sc as plsc

N = 16384
H = 200
L = 16
W = 128


@functools.lru_cache(maxsize=None)
def _build():
    info = plsc.get_sparse_core_info()
    NC, NS = info.num_cores, info.num_subcores
    NW = NC * NS
    R = N // NW
    TCOLS = R // W
    G = W // L

    mesh = plsc.VectorSubcoreMesh(core_axis_name="c", subcore_axis_name="s")

    @functools.partial(
        pl.kernel,
        mesh=mesh,
        out_type=jax.ShapeDtypeStruct((H, N), jnp.float32),
        compiler_params=pltpu.CompilerParams(needs_layout_passes=False),
        scratch_types=[
            pltpu.VMEM((R,), jnp.int32),
            pltpu.VMEM((R,), jnp.float32),
            pltpu.VMEM((H, W), jnp.float32),
            pltpu.VMEM((H, W), jnp.float32),
            pltpu.VMEM((H, W), jnp.float32),
            pltpu.VMEM((H, W), jnp.float32),
            pltpu.SemaphoreType.DMA,
            pltpu.SemaphoreType.DMA,
            pltpu.SemaphoreType.DMA,
            pltpu.SemaphoreType.DMA,
        ],
    )
    def sc_kernel(
        x_hbm, bufT_hbm, i_hbm, outT_hbm,
        i_v, x_v, in0, in1, out0, out1, is0, is1, os0, os1,
    ):
        wid = lax.axis_index("s") * NC + lax.axis_index("c")
        base = wid * R
        pltpu.sync_copy(i_hbm.at[pl.ds(base, R)], i_v)
        pltpu.sync_copy(x_hbm.at[pl.ds(base, R)], x_v)
        lane = lax.iota(jnp.int32, L)
        ins, outs = [in0, in1], [out0, out1]
        isems, osems = [is0, is1], [os0, os1]

        def copy_in(t, dst, sem):
            return pltpu.make_async_copy(
                bufT_hbm.at[:, pl.ds(base + t * W, W)], dst, sem
            )

        def copy_out(t, src, sem):
            return pltpu.make_async_copy(
                src, outT_hbm.at[:, pl.ds(base + t * W, W)], sem
            )

        def compute(t, in_ref, out_ref):
            nls = []
            hs = []
            for g in range(G):
                off = t * W + g * L
                ivec = i_v[pl.ds(off, L)]
                xvec = x_v[pl.ds(off, L)]
                nl = g * L + lane
                plsc.store_scatter(in_ref, [ivec, nl], xvec)
                nls.append(nl)
                hs.append(ivec.astype(jnp.uint32))

            wrap = jnp.uint32(H - 1)

            def body(k2, hs):
                for step in range(2):
                    k = k2 * 2 + step
                    vals = [
                        plsc.load_gather(
                            in_ref, [hs[g].astype(jnp.int32), nls[g]]
                        )
                        for g in range(G)
                    ]
                    for g in range(G):
                        out_ref[k, pl.ds(g * L, L)] = vals[g]
                    hs = tuple(jnp.minimum(hs[g] - 1, wrap) for g in range(G))
                return hs

            lax.fori_loop(0, H // 2, body, tuple(hs))

        copy_in(0, ins[0], isems[0]).start()
        for t in range(TCOLS):
            b = t % 2
            if t + 1 < TCOLS:
                copy_in(t + 1, ins[1 - b], isems[1 - b]).start()
            copy_in(t, ins[b], isems[b]).wait()
            if t >= 2:
                copy_out(t - 2, outs[b], osems[b]).wait()
            compute(t, ins[b], outs[b])
            copy_out(t, outs[b], osems[b]).start()
        copy_out(TCOLS - 2, outs[(TCOLS - 2) % 2], osems[(TCOLS - 2) % 2]).wait()
        copy_out(TCOLS - 1, outs[(TCOLS - 1) % 2], osems[(TCOLS - 1) % 2]).wait()

    return sc_kernel


def kernel(x, buffer, i):
    out_t = _build()(x.reshape(-1), buffer.T, i.astype(jnp.int32))
    return out_t.T


# speedup vs baseline: 16.3462x; 1.0448x over previous
import functools

import jax
import jax.numpy as jnp
from jax import lax
from jax.experimental import pallas as pl
from jax.experimental.pallas import tpu as pltpu
from jax.experimental.pallas import tpu_sc as plsc

N = 16384
H = 200
L = 16
W = 128


@functools.lru_cache(maxsize=None)
def _build():
    info = plsc.get_sparse_core_info()
    NC, NS = info.num_cores, info.num_subcores
    NW = NC * NS
    R = N // NW
    TCOLS = R // W
    G = W // L

    mesh = plsc.VectorSubcoreMesh(core_axis_name="c", subcore_axis_name="s")

    @functools.partial(
        pl.kernel,
        mesh=mesh,
        out_type=jax.ShapeDtypeStruct((H, N), jnp.float32),
        compiler_params=pltpu.CompilerParams(needs_layout_passes=False),
        scratch_types=[
            pltpu.VMEM((R,), jnp.int32),
            pltpu.VMEM((R,), jnp.float32),
            pltpu.VMEM((H, W), jnp.float32),
            pltpu.VMEM((H, W), jnp.float32),
            pltpu.VMEM((H, W), jnp.float32),
            pltpu.VMEM((H, W), jnp.float32),
            pltpu.SemaphoreType.DMA,
            pltpu.SemaphoreType.DMA,
            pltpu.SemaphoreType.DMA,
            pltpu.SemaphoreType.DMA,
            pltpu.SemaphoreType.DMA,
        ],
    )
    def sc_kernel(
        x_hbm, bufT_hbm, i_hbm, outT_hbm,
        i_v, x_v, in0, in1, out0, out1, is0, is1, os0, os1, ps0,
    ):
        wid = lax.axis_index("s") * NC + lax.axis_index("c")
        base = wid * R
        iv_cp = pltpu.make_async_copy(i_hbm.at[pl.ds(base, R)], i_v, ps0)
        xv_cp = pltpu.make_async_copy(x_hbm.at[pl.ds(base, R)], x_v, ps0)
        iv_cp.start()
        xv_cp.start()
        lane = lax.iota(jnp.int32, L)
        ins, outs = [in0, in1], [out0, out1]
        isems, osems = [is0, is1], [os0, os1]

        def copy_in(t, dst, sem):
            return pltpu.make_async_copy(
                bufT_hbm.at[:, pl.ds(base + t * W, W)], dst, sem
            )

        def copy_out(t, src, sem):
            return pltpu.make_async_copy(
                src, outT_hbm.at[:, pl.ds(base + t * W, W)], sem
            )

        def compute(t, in_ref, out_ref):
            nls = []
            hs = []
            for g in range(G):
                off = t * W + g * L
                ivec = i_v[pl.ds(off, L)]
                xvec = x_v[pl.ds(off, L)]
                nl = g * L + lane
                plsc.store_scatter(in_ref, [ivec, nl], xvec)
                nls.append(nl)
                hs.append(ivec.astype(jnp.uint32))

            wrap = jnp.uint32(H - 1)

            def body(k2, hs):
                for step in range(2):
                    k = k2 * 2 + step
                    vals = [
                        plsc.load_gather(
                            in_ref, [hs[g].astype(jnp.int32), nls[g]]
                        )
                        for g in range(G)
                    ]
                    for g in range(G):
                        out_ref[k, pl.ds(g * L, L)] = vals[g]
                    hs = tuple(jnp.minimum(hs[g] - 1, wrap) for g in range(G))
                return hs

            lax.fori_loop(0, H // 2, body, tuple(hs))

        copy_in(0, ins[0], isems[0]).start()
        iv_cp.wait()
        xv_cp.wait()
        for t in range(TCOLS):
            b = t % 2
            if t + 1 < TCOLS:
                copy_in(t + 1, ins[1 - b], isems[1 - b]).start()
            copy_in(t, ins[b], isems[b]).wait()
            if t >= 2:
                copy_out(t - 2, outs[b], osems[b]).wait()
            compute(t, ins[b], outs[b])
            copy_out(t, outs[b], osems[b]).start()
        copy_out(TCOLS - 2, outs[(TCOLS - 2) % 2], osems[(TCOLS - 2) % 2]).wait()
        copy_out(TCOLS - 1, outs[(TCOLS - 1) % 2], osems[(TCOLS - 1) % 2]).wait()

    return sc_kernel


def kernel(x, buffer, i):
    out_t = _build()(x.reshape(-1), buffer.T, i.astype(jnp.int32))
    return out_t.T
